# baseline (speedup 1.0000x reference)
import functools

import jax
import jax.numpy as jnp
from jax import lax
from jax.experimental import pallas as pl
from jax.experimental.pallas import tpu as pltpu
from jax.experimental.pallas import tpu_sc as plsc

B = 4096
D = 1024
P = 8192
K = 8
EPS = 1e-8

TB = 512
TP = 1024
NB = B // TB
NP = P // TP
CHUNK = 128
SLAB = 2048
PREP_R = 512


def _prep_body(x_ref, hi_ref, lo_ref):
    x = x_ref[...]
    n = jnp.sqrt(jnp.sum(x * x, axis=1, keepdims=True))
    xn = x / jnp.maximum(n, EPS)
    hi = xn.astype(jnp.bfloat16)
    lo = (xn - hi.astype(jnp.float32)).astype(jnp.bfloat16)
    hi_ref[...] = hi
    lo_ref[...] = lo


def _normalize_split(x):
    rows = x.shape[0]
    return pl.pallas_call(
        _prep_body,
        grid=(rows // PREP_R,),
        in_specs=[pl.BlockSpec((PREP_R, D), lambda i: (i, 0))],
        out_specs=[
            pl.BlockSpec((PREP_R, D), lambda i: (i, 0)),
            pl.BlockSpec((PREP_R, D), lambda i: (i, 0)),
        ],
        out_shape=[
            jax.ShapeDtypeStruct((rows, D), jnp.bfloat16),
            jax.ShapeDtypeStruct((rows, D), jnp.bfloat16),
        ],
    )(x)


_SORT8 = [(0, 1), (2, 3), (4, 5), (6, 7),
          (0, 2), (1, 3), (4, 6), (5, 7),
          (1, 2), (5, 6),
          (0, 4), (1, 5), (2, 6), (3, 7),
          (2, 4), (3, 5),
          (1, 2), (3, 4), (5, 6)]
_MERGE8 = [(0, 4), (1, 5), (2, 6), (3, 7),
           (0, 2), (1, 3), (4, 6), (5, 7),
           (0, 1), (2, 3), (4, 5), (6, 7)]


def _ce(v, ix, a, b):
    c = v[a] <= v[b]
    va = jnp.where(c, v[a], v[b])
    vb = jnp.where(c, v[b], v[a])
    ia = jnp.where(c, ix[a], ix[b])
    ib = jnp.where(c, ix[b], ix[a])
    v[a], v[b] = va, vb
    ix[a], ix[b] = ia, ib


def _merge_keep8(av, ai, bv, bi):
    v, ix = [], []
    for p in range(K):
        c = av[p] <= bv[K - 1 - p]
        v.append(jnp.where(c, av[p], bv[K - 1 - p]))
        ix.append(jnp.where(c, ai[p], bi[K - 1 - p]))
    for a, b in _MERGE8:
        _ce(v, ix, a, b)
    return v, ix


def _topk8(v, rows):
    G = P // 16
    giota = lax.broadcasted_iota(jnp.int32, (rows, G), 1)
    v = list(v)
    ix = [giota + t * G for t in range(16)]
    for a, b in _SORT8:
        _ce(v, ix, a, b)
        _ce(v, ix, a + 8, b + 8)
    v8, ix8 = _merge_keep8(v[0:8], ix[0:8], v[8:16], ix[8:16])
    w = G // 2
    while w >= 128:
        av = [x[:, :w] for x in v8]
        ai = [x[:, :w] for x in ix8]
        bv = [x[:, w:2 * w] for x in v8]
        bi = [x[:, w:2 * w] for x in ix8]
        v8, ix8 = _merge_keep8(av, ai, bv, bi)
        w //= 2
    cv = jnp.concatenate(v8, axis=1)
    ci = jnp.concatenate(ix8, axis=1)
    ncand = cv.shape[1]
    cpos = lax.broadcasted_iota(jnp.int32, (rows, ncand), 1)
    vals_l, idxs_l = [], []
    for t in range(K):
        mv = jnp.min(cv, axis=1, keepdims=True)
        pm = jnp.min(jnp.where(cv == mv, cpos, ncand), axis=1, keepdims=True)
        vals_l.append(mv)
        idxs_l.append(jnp.min(jnp.where(cpos == pm, ci, P), axis=1,
                              keepdims=True))
        if t < K - 1:
            cv = jnp.where(cpos == pm, jnp.float32(jnp.inf), cv)
    vals = jnp.concatenate(vals_l, axis=1)
    idxs = jnp.concatenate(idxs_l, axis=1)
    return vals, idxs


def _knorm_body(x_ref, n_ref):
    x = x_ref[...]
    n = jnp.sqrt(jnp.sum(x * x, axis=1))
    n_ref[...] = jnp.maximum(n, EPS)[None, :]


def _key_norms(keys):
    return pl.pallas_call(
        _knorm_body,
        grid=(P // PREP_R,),
        in_specs=[pl.BlockSpec((PREP_R, D), lambda i: (i, 0))],
        out_specs=pl.BlockSpec((1, PREP_R), lambda i: (0, i)),
        out_shape=jax.ShapeDtypeStruct((1, P), jnp.float32),
    )(keys)


def _tc_body(q_ref, keys_ref, kn_ref, idx_ref, ent_ref, ssum_ref, scores):
    i = pl.program_id(0)
    j = pl.program_id(1)
    qb = q_ref[...]
    kb = keys_ref[...]
    dn = (((1,), (1,)), ((), ()))
    dots = lax.dot_general(qb, kb, dn, preferred_element_type=jnp.float32)
    kn_row = kn_ref[...]
    qn = jnp.maximum(jnp.sqrt(jnp.sum(qb * qb, axis=1)), EPS)
    dots = dots / (qn[:, None] * kn_row)
    scores[:, pl.ds(j * TP, TP)] = 1.0 - dots

    @pl.when(j == NP - 1)
    def _finish():
        ent_tot = jnp.float32(0.0)
        score_tot = jnp.float32(0.0)
        G = P // 16
        for c in range(TB // CHUNK):
            s = scores[pl.ds(c * CHUNK, CHUNK), :]
            vs = [s[:, t * G:(t + 1) * G] for t in range(16)]
            mm = vs[0]
            for t in range(1, 16):
                mm = jnp.maximum(mm, vs[t])
            m = jnp.max(mm, axis=1, keepdims=True)
            se_p = jnp.exp(vs[0] - m)
            sx_p = vs[0] * se_p
            for t in range(1, 16):
                e_t = jnp.exp(vs[t] - m)
                se_p = se_p + e_t
                sx_p = sx_p + vs[t] * e_t
            se = jnp.sum(se_p, axis=1, keepdims=True)
            sx = jnp.sum(sx_p, axis=1, keepdims=True)
            ent = m[:, 0] + jnp.log(se[:, 0]) - sx[:, 0] / se[:, 0]
            ent_tot = ent_tot + jnp.sum(ent)
            vals, idxs = _topk8(vs, CHUNK)
            score_tot = score_tot + jnp.sum(vals)
            idx_ref[pl.ds(c * CHUNK, CHUNK), :] = idxs
        ent_ref[i] = ent_tot
        ssum_ref[i] = score_tot


def _tc_scores_topk(q, keys, kn, off_rows, rows):
    ob = off_rows // TB
    return pl.pallas_call(
        _tc_body,
        grid=(rows // TB, NP),
        in_specs=[
            pl.BlockSpec((TB, D), lambda i, j: (i + ob, 0)),
            pl.BlockSpec((TP, D), lambda i, j: (j, 0)),
            pl.BlockSpec((1, TP), lambda i, j: (0, j)),
        ],
        out_specs=[
            pl.BlockSpec((TB, K), lambda i, j: (i, 0)),
            pl.BlockSpec(memory_space=pltpu.SMEM),
            pl.BlockSpec(memory_space=pltpu.SMEM),
        ],
        out_shape=[
            jax.ShapeDtypeStruct((rows, K), jnp.int32),
            jax.ShapeDtypeStruct((rows // TB,), jnp.float32),
            jax.ShapeDtypeStruct((rows // TB,), jnp.float32),
        ],
        scratch_shapes=[pltpu.VMEM((TB, P), jnp.float32)],
        compiler_params=pltpu.CompilerParams(
            dimension_semantics=("arbitrary", "arbitrary")),
    )(q, keys, kn)


_SC_NC = 2
_SC_NS = 16
_NW = _SC_NC * _SC_NS
_CB = 4


def _sc_gather_mean(idx_flat, ppg2d, prompt, base_row, rows):
    _PER_W = rows // _NW
    _NCHUNK = _PER_W // _CB
    mesh = plsc.VectorSubcoreMesh(core_axis_name="c", subcore_axis_name="s")

    @functools.partial(
        pl.kernel,
        mesh=mesh,
        out_type=jax.ShapeDtypeStruct((rows, D), jnp.float32),
        scratch_types=[
            pltpu.VMEM((2, _CB * K), jnp.int32),
            pltpu.VMEM((2, _CB * K, D), jnp.float32),
            pltpu.VMEM((2, _CB, D), jnp.float32),
            pltpu.VMEM((2, _CB, D), jnp.float32),
            pltpu.SemaphoreType.DMA,
            pltpu.SemaphoreType.DMA,
            pltpu.SemaphoreType.DMA,
            pltpu.SemaphoreType.DMA,
        ],
    )
    def sc_kernel(idx_hbm, ppg_hbm, prompt_hbm, out_hbm,
                  idx_v, rows_v, ppg_v, out_v, semg0, semg1, semo0, semo1):
        wid = lax.axis_index("s") * _SC_NC + lax.axis_index("c")
        semg = (semg0, semg1)
        semo = (semo0, semo1)
        g_handles = {}
        out_handles = {}

        def start(c):
            b = c % 2
            base = wid * _PER_W + c * _CB
            pltpu.sync_copy(idx_hbm.at[pl.ds(base * K, _CB * K)],
                            idx_v.at[b])
            g_handles[c] = pltpu.async_copy(prompt_hbm.at[idx_v.at[b]],
                                            rows_v.at[b], semg[b])
            pltpu.sync_copy(ppg_hbm.at[pl.ds(base_row + base, _CB)],
                            ppg_v.at[b])

        start(0)
        for c in range(_NCHUNK):
            b = c % 2
            base = wid * _PER_W + c * _CB
            if c + 1 < _NCHUNK:
                start(c + 1)
            g_handles[c].wait()
            if c >= 2:
                out_handles[c - 2].wait()

            def dbody(dd, c2, b=b):
                off = dd * 16
                for r in range(_CB):
                    acc = rows_v[b, r * K + 0, pl.ds(off, 16)]
                    for k in range(1, K):
                        acc = acc + rows_v[b, r * K + k, pl.ds(off, 16)]
                    out_v[b, r, pl.ds(off, 16)] = (
                        ppg_v[b, r, pl.ds(off, 16)] + acc * (1.0 / K))
                return c2

            lax.fori_loop(0, D // 16, dbody, 0)
            out_handles[c] = pltpu.async_copy(
                out_v.at[b], out_hbm.at[pl.ds(base, _CB)], semo[b])
        out_handles[_NCHUNK - 2].wait()
        out_handles[_NCHUNK - 1].wait()

    return sc_kernel(idx_flat, ppg2d, prompt)


def kernel(ppg, mode, group_labels, keys, prompt, group_table):
    q = ppg[:, 0, :]
    kn = _key_norms(keys)
    parts, ents, ssums = [], [], []
    for s in range(B // SLAB):
        idx_s, ent_s, ssum_s = _tc_scores_topk(q, keys, kn, s * SLAB, SLAB)
        parts.append(_sc_gather_mean(idx_s.reshape(SLAB * K), q, prompt,
                                     s * SLAB, SLAB))
        ents.append(ent_s)
        ssums.append(ssum_s)
    prompted = jnp.concatenate(parts, axis=0)[:, None, :]
    score_mean = jnp.sum(jnp.stack(ssums)) / (B * K)
    entropy = jnp.sum(jnp.stack(ents)) / B
    return (prompted, score_mean, entropy)


# speedup vs baseline: 1.0520x; 1.0520x over previous
import functools

import jax
import jax.numpy as jnp
from jax import lax
from jax.experimental import pallas as pl
from jax.experimental.pallas import tpu as pltpu
from jax.experimental.pallas import tpu_sc as plsc

B = 4096
D = 1024
P = 8192
K = 8
EPS = 1e-8

TB = 512
TP = 1024
NB = B // TB
NP = P // TP
CHUNK = 128
SLAB = 2048
PREP_R = 512


def _prep_body(x_ref, hi_ref, lo_ref):
    x = x_ref[...]
    n = jnp.sqrt(jnp.sum(x * x, axis=1, keepdims=True))
    xn = x / jnp.maximum(n, EPS)
    hi = xn.astype(jnp.bfloat16)
    lo = (xn - hi.astype(jnp.float32)).astype(jnp.bfloat16)
    hi_ref[...] = hi
    lo_ref[...] = lo


def _normalize_split(x):
    rows = x.shape[0]
    return pl.pallas_call(
        _prep_body,
        grid=(rows // PREP_R,),
        in_specs=[pl.BlockSpec((PREP_R, D), lambda i: (i, 0))],
        out_specs=[
            pl.BlockSpec((PREP_R, D), lambda i: (i, 0)),
            pl.BlockSpec((PREP_R, D), lambda i: (i, 0)),
        ],
        out_shape=[
            jax.ShapeDtypeStruct((rows, D), jnp.bfloat16),
            jax.ShapeDtypeStruct((rows, D), jnp.bfloat16),
        ],
    )(x)


_SORT8 = [(0, 1), (2, 3), (4, 5), (6, 7),
          (0, 2), (1, 3), (4, 6), (5, 7),
          (1, 2), (5, 6),
          (0, 4), (1, 5), (2, 6), (3, 7),
          (2, 4), (3, 5),
          (1, 2), (3, 4), (5, 6)]
_MERGE8 = [(0, 4), (1, 5), (2, 6), (3, 7),
           (0, 2), (1, 3), (4, 6), (5, 7),
           (0, 1), (2, 3), (4, 5), (6, 7)]


def _ce(v, ix, a, b):
    c = v[a] <= v[b]
    va = jnp.where(c, v[a], v[b])
    vb = jnp.where(c, v[b], v[a])
    ia = jnp.where(c, ix[a], ix[b])
    ib = jnp.where(c, ix[b], ix[a])
    v[a], v[b] = va, vb
    ix[a], ix[b] = ia, ib


def _merge_keep8(av, ai, bv, bi):
    v, ix = [], []
    for p in range(K):
        c = av[p] <= bv[K - 1 - p]
        v.append(jnp.where(c, av[p], bv[K - 1 - p]))
        ix.append(jnp.where(c, ai[p], bi[K - 1 - p]))
    for a, b in _MERGE8:
        _ce(v, ix, a, b)
    return v, ix


def _topk8(v, rows):
    G = P // 16
    giota = lax.broadcasted_iota(jnp.int32, (rows, G), 1)
    v = list(v)
    ix = [giota + t * G for t in range(16)]
    for a, b in _SORT8:
        _ce(v, ix, a, b)
        _ce(v, ix, a + 8, b + 8)
    v8, ix8 = _merge_keep8(v[0:8], ix[0:8], v[8:16], ix[8:16])
    w = G // 2
    while w >= 128:
        av = [x[:, :w] for x in v8]
        ai = [x[:, :w] for x in ix8]
        bv = [x[:, w:2 * w] for x in v8]
        bi = [x[:, w:2 * w] for x in ix8]
        v8, ix8 = _merge_keep8(av, ai, bv, bi)
        w //= 2
    cv = jnp.concatenate(v8, axis=1)
    ci = jnp.concatenate(ix8, axis=1)
    ncand = cv.shape[1]
    cpos = lax.broadcasted_iota(jnp.int32, (rows, ncand), 1)
    vals_l, idxs_l = [], []
    for t in range(K):
        mv = jnp.min(cv, axis=1, keepdims=True)
        pm = jnp.min(jnp.where(cv == mv, cpos, ncand), axis=1, keepdims=True)
        vals_l.append(mv)
        idxs_l.append(jnp.min(jnp.where(cpos == pm, ci, P), axis=1,
                              keepdims=True))
        if t < K - 1:
            cv = jnp.where(cpos == pm, jnp.float32(jnp.inf), cv)
    vals = jnp.concatenate(vals_l, axis=1)
    idxs = jnp.concatenate(idxs_l, axis=1)
    return vals, idxs


def _knorm_body(x_ref, n_ref):
    x = x_ref[...]
    n = jnp.sqrt(jnp.sum(x * x, axis=1))
    n_ref[...] = jnp.maximum(n, EPS)[None, :]


def _key_norms(keys):
    return pl.pallas_call(
        _knorm_body,
        grid=(P // PREP_R,),
        in_specs=[pl.BlockSpec((PREP_R, D), lambda i: (i, 0))],
        out_specs=pl.BlockSpec((1, PREP_R), lambda i: (0, i)),
        out_shape=jax.ShapeDtypeStruct((1, P), jnp.float32),
    )(keys)


def _tc_body(q_ref, keys_ref, kn_ref, idx_ref, ent_ref, ssum_ref, scores):
    i = pl.program_id(0)
    j = pl.program_id(1)
    qb = q_ref[...]
    kb = keys_ref[...]
    dn = (((1,), (1,)), ((), ()))
    dots = lax.dot_general(qb, kb, dn, preferred_element_type=jnp.float32)
    kn_row = kn_ref[...]
    qn = jnp.maximum(jnp.sqrt(jnp.sum(qb * qb, axis=1)), EPS)
    dots = dots / (qn[:, None] * kn_row)
    scores[:, pl.ds(j * TP, TP)] = 1.0 - dots

    @pl.when(j == NP - 1)
    def _finish():
        ent_tot = jnp.float32(0.0)
        score_tot = jnp.float32(0.0)
        G = P // 16
        for c in range(TB // CHUNK):
            s = scores[pl.ds(c * CHUNK, CHUNK), :]
            vs = [s[:, t * G:(t + 1) * G] for t in range(16)]
            mm = vs[0]
            for t in range(1, 16):
                mm = jnp.maximum(mm, vs[t])
            m = jnp.max(mm, axis=1, keepdims=True)
            se_p = jnp.exp(vs[0] - m)
            sx_p = vs[0] * se_p
            for t in range(1, 16):
                e_t = jnp.exp(vs[t] - m)
                se_p = se_p + e_t
                sx_p = sx_p + vs[t] * e_t
            se = jnp.sum(se_p, axis=1, keepdims=True)
            sx = jnp.sum(sx_p, axis=1, keepdims=True)
            ent = m[:, 0] + jnp.log(se[:, 0]) - sx[:, 0] / se[:, 0]
            ent_tot = ent_tot + jnp.sum(ent)
            vals, idxs = _topk8(vs, CHUNK)
            score_tot = score_tot + jnp.sum(vals)
            idx_ref[pl.ds(c * CHUNK, CHUNK), :] = idxs
        ent_ref[i] = ent_tot
        ssum_ref[i] = score_tot


def _tc_scores_topk(q, keys, kn, off_rows, rows):
    ob = off_rows // TB
    return pl.pallas_call(
        _tc_body,
        grid=(rows // TB, NP),
        in_specs=[
            pl.BlockSpec((TB, D), lambda i, j: (i + ob, 0)),
            pl.BlockSpec((TP, D), lambda i, j: (j, 0)),
            pl.BlockSpec((1, TP), lambda i, j: (0, j)),
        ],
        out_specs=[
            pl.BlockSpec((TB, K), lambda i, j: (i, 0)),
            pl.BlockSpec(memory_space=pltpu.SMEM),
            pl.BlockSpec(memory_space=pltpu.SMEM),
        ],
        out_shape=[
            jax.ShapeDtypeStruct((rows, K), jnp.int32),
            jax.ShapeDtypeStruct((rows // TB,), jnp.float32),
            jax.ShapeDtypeStruct((rows // TB,), jnp.float32),
        ],
        scratch_shapes=[pltpu.VMEM((TB, P), jnp.float32)],
        compiler_params=pltpu.CompilerParams(
            dimension_semantics=("arbitrary", "arbitrary")),
    )(q, keys, kn)


_SC_NC = 2
_SC_NS = 16
_NW = _SC_NC * _SC_NS
_CB = 8


def _sc_gather_mean(idx_flat, ppg2d, prompt, base_row, rows):
    _PER_W = rows // _NW
    _NCHUNK = _PER_W // _CB
    mesh = plsc.VectorSubcoreMesh(core_axis_name="c", subcore_axis_name="s")

    @functools.partial(
        pl.kernel,
        mesh=mesh,
        out_type=jax.ShapeDtypeStruct((rows, D), jnp.float32),
        scratch_types=[
            pltpu.VMEM((_CB * K,), jnp.int32),
            pltpu.VMEM((_CB * K, D), jnp.float32),
            pltpu.VMEM((_CB, D), jnp.float32),
            pltpu.VMEM((2, _CB, D), jnp.float32),
            pltpu.SemaphoreType.DMA,
            pltpu.SemaphoreType.DMA,
            pltpu.SemaphoreType.DMA,
        ],
    )
    def sc_kernel(idx_hbm, ppg_hbm, prompt_hbm, out_hbm,
                  idx_v, rows_v, ppg_v, out_v, semg, semg2, semo):
        wid = lax.axis_index("s") * _SC_NC + lax.axis_index("c")
        out_handles = {}

        H = _CB // 2

        def chunk(c):
            b = c % 2
            base = wid * _PER_W + c * _CB
            pltpu.sync_copy(idx_hbm.at[pl.ds(base * K, _CB * K)], idx_v)
            g0 = pltpu.async_copy(prompt_hbm.at[idx_v.at[pl.ds(0, H * K)]],
                                  rows_v.at[pl.ds(0, H * K)], semg)
            g1 = pltpu.async_copy(
                prompt_hbm.at[idx_v.at[pl.ds(H * K, H * K)]],
                rows_v.at[pl.ds(H * K, H * K)], semg2)
            pltpu.sync_copy(ppg_hbm.at[pl.ds(base_row + base, _CB)], ppg_v)

            def half_body(r0):
                def dbody(dd, c2):
                    off = dd * 16
                    for r in range(r0, r0 + H):
                        acc = rows_v[r * K + 0, pl.ds(off, 16)]
                        for k in range(1, K):
                            acc = acc + rows_v[r * K + k, pl.ds(off, 16)]
                        out_v[b, r, pl.ds(off, 16)] = (
                            ppg_v[r, pl.ds(off, 16)] + acc * (1.0 / K))
                    return c2

                lax.fori_loop(0, D // 16, dbody, 0)

            g0.wait()
            half_body(0)
            g1.wait()
            half_body(H)
            out_handles[c] = pltpu.async_copy(
                out_v.at[b], out_hbm.at[pl.ds(base, _CB)], semo)

        for c in range(_NCHUNK):
            if c >= 2:
                out_handles[c - 2].wait()
            chunk(c)
        out_handles[_NCHUNK - 2].wait()
        out_handles[_NCHUNK - 1].wait()

    return sc_kernel(idx_flat, ppg2d, prompt)


def kernel(ppg, mode, group_labels, keys, prompt, group_table):
    q = ppg[:, 0, :]
    kn = _key_norms(keys)
    parts, ents, ssums = [], [], []
    for s in range(B // SLAB):
        idx_s, ent_s, ssum_s = _tc_scores_topk(q, keys, kn, s * SLAB, SLAB)
        parts.append(_sc_gather_mean(idx_s.reshape(SLAB * K), q, prompt,
                                     s * SLAB, SLAB))
        ents.append(ent_s)
        ssums.append(ssum_s)
    prompted = jnp.concatenate(parts, axis=0)[:, None, :]
    score_mean = jnp.sum(jnp.stack(ssums)) / (B * K)
    entropy = jnp.sum(jnp.stack(ents)) / B
    return (prompted, score_mean, entropy)


# speedup vs baseline: 1.0636x; 1.0111x over previous
import functools

import jax
import jax.numpy as jnp
from jax import lax
from jax.experimental import pallas as pl
from jax.experimental.pallas import tpu as pltpu
from jax.experimental.pallas import tpu_sc as plsc

B = 4096
D = 1024
P = 8192
K = 8
EPS = 1e-8

TB = 512
TP = 1024
NB = B // TB
NP = P // TP
CHUNK = 256
SLAB = 2048
PREP_R = 512


def _prep_body(x_ref, hi_ref, lo_ref):
    x = x_ref[...]
    n = jnp.sqrt(jnp.sum(x * x, axis=1, keepdims=True))
    xn = x / jnp.maximum(n, EPS)
    hi = xn.astype(jnp.bfloat16)
    lo = (xn - hi.astype(jnp.float32)).astype(jnp.bfloat16)
    hi_ref[...] = hi
    lo_ref[...] = lo


def _normalize_split(x):
    rows = x.shape[0]
    return pl.pallas_call(
        _prep_body,
        grid=(rows // PREP_R,),
        in_specs=[pl.BlockSpec((PREP_R, D), lambda i: (i, 0))],
        out_specs=[
            pl.BlockSpec((PREP_R, D), lambda i: (i, 0)),
            pl.BlockSpec((PREP_R, D), lambda i: (i, 0)),
        ],
        out_shape=[
            jax.ShapeDtypeStruct((rows, D), jnp.bfloat16),
            jax.ShapeDtypeStruct((rows, D), jnp.bfloat16),
        ],
    )(x)


_SORT8 = [(0, 1), (2, 3), (4, 5), (6, 7),
          (0, 2), (1, 3), (4, 6), (5, 7),
          (1, 2), (5, 6),
          (0, 4), (1, 5), (2, 6), (3, 7),
          (2, 4), (3, 5),
          (1, 2), (3, 4), (5, 6)]
_MERGE8 = [(0, 4), (1, 5), (2, 6), (3, 7),
           (0, 2), (1, 3), (4, 6), (5, 7),
           (0, 1), (2, 3), (4, 5), (6, 7)]


def _ce(v, ix, a, b):
    c = v[a] <= v[b]
    va = jnp.where(c, v[a], v[b])
    vb = jnp.where(c, v[b], v[a])
    ia = jnp.where(c, ix[a], ix[b])
    ib = jnp.where(c, ix[b], ix[a])
    v[a], v[b] = va, vb
    ix[a], ix[b] = ia, ib


def _merge_keep8(av, ai, bv, bi):
    v, ix = [], []
    for p in range(K):
        c = av[p] <= bv[K - 1 - p]
        v.append(jnp.where(c, av[p], bv[K - 1 - p]))
        ix.append(jnp.where(c, ai[p], bi[K - 1 - p]))
    for a, b in _MERGE8:
        _ce(v, ix, a, b)
    return v, ix


def _topk8(v, rows):
    G = P // 16
    giota = lax.broadcasted_iota(jnp.int32, (rows, G), 1)
    v = list(v)
    ix = [giota + t * G for t in range(16)]
    for a, b in _SORT8:
        _ce(v, ix, a, b)
        _ce(v, ix, a + 8, b + 8)
    v8, ix8 = _merge_keep8(v[0:8], ix[0:8], v[8:16], ix[8:16])
    w = G // 2
    while w >= 128:
        av = [x[:, :w] for x in v8]
        ai = [x[:, :w] for x in ix8]
        bv = [x[:, w:2 * w] for x in v8]
        bi = [x[:, w:2 * w] for x in ix8]
        v8, ix8 = _merge_keep8(av, ai, bv, bi)
        w //= 2
    cv = jnp.concatenate(v8, axis=1)
    ci = jnp.concatenate(ix8, axis=1)
    ncand = cv.shape[1]
    cpos = lax.broadcasted_iota(jnp.int32, (rows, ncand), 1)
    vals_l, idxs_l = [], []
    for t in range(K):
        mv = jnp.min(cv, axis=1, keepdims=True)
        pm = jnp.min(jnp.where(cv == mv, cpos, ncand), axis=1, keepdims=True)
        vals_l.append(mv)
        idxs_l.append(jnp.min(jnp.where(cpos == pm, ci, P), axis=1,
                              keepdims=True))
        if t < K - 1:
            cv = jnp.where(cpos == pm, jnp.float32(jnp.inf), cv)
    vals = jnp.concatenate(vals_l, axis=1)
    idxs = jnp.concatenate(idxs_l, axis=1)
    return vals, idxs


def _knorm_body(x_ref, n_ref):
    x = x_ref[...]
    n = jnp.sqrt(jnp.sum(x * x, axis=1))
    n_ref[...] = jnp.maximum(n, EPS)[None, :]


def _key_norms(keys):
    return pl.pallas_call(
        _knorm_body,
        grid=(P // PREP_R,),
        in_specs=[pl.BlockSpec((PREP_R, D), lambda i: (i, 0))],
        out_specs=pl.BlockSpec((1, PREP_R), lambda i: (0, i)),
        out_shape=jax.ShapeDtypeStruct((1, P), jnp.float32),
    )(keys)


def _tc_body(q_ref, keys_ref, kn_ref, idx_ref, ent_ref, ssum_ref, scores):
    i = pl.program_id(0)
    j = pl.program_id(1)
    qb = q_ref[...]
    kb = keys_ref[...]
    dn = (((1,), (1,)), ((), ()))
    dots = lax.dot_general(qb, kb, dn, preferred_element_type=jnp.float32)
    kn_row = kn_ref[...]
    qn = jnp.maximum(jnp.sqrt(jnp.sum(qb * qb, axis=1)), EPS)
    dots = dots / (qn[:, None] * kn_row)
    scores[:, pl.ds(j * TP, TP)] = 1.0 - dots

    @pl.when(j == NP - 1)
    def _finish():
        ent_tot = jnp.float32(0.0)
        score_tot = jnp.float32(0.0)
        G = P // 16
        for c in range(TB // CHUNK):
            s = scores[pl.ds(c * CHUNK, CHUNK), :]
            vs = [s[:, t * G:(t + 1) * G] for t in range(16)]
            mm = vs[0]
            for t in range(1, 16):
                mm = jnp.maximum(mm, vs[t])
            m = jnp.max(mm, axis=1, keepdims=True)
            se_p = jnp.exp(vs[0] - m)
            sx_p = vs[0] * se_p
            for t in range(1, 16):
                e_t = jnp.exp(vs[t] - m)
                se_p = se_p + e_t
                sx_p = sx_p + vs[t] * e_t
            se = jnp.sum(se_p, axis=1, keepdims=True)
            sx = jnp.sum(sx_p, axis=1, keepdims=True)
            ent = m[:, 0] + jnp.log(se[:, 0]) - sx[:, 0] / se[:, 0]
            ent_tot = ent_tot + jnp.sum(ent)
            vals, idxs = _topk8(vs, CHUNK)
            score_tot = score_tot + jnp.sum(vals)
            idx_ref[pl.ds(c * CHUNK, CHUNK), :] = idxs
        ent_ref[i] = ent_tot
        ssum_ref[i] = score_tot


def _tc_scores_topk(q, keys, kn, off_rows, rows):
    ob = off_rows // TB
    return pl.pallas_call(
        _tc_body,
        grid=(rows // TB, NP),
        in_specs=[
            pl.BlockSpec((TB, D), lambda i, j: (i + ob, 0)),
            pl.BlockSpec((TP, D), lambda i, j: (j, 0)),
            pl.BlockSpec((1, TP), lambda i, j: (0, j)),
        ],
        out_specs=[
            pl.BlockSpec((TB, K), lambda i, j: (i, 0)),
            pl.BlockSpec(memory_space=pltpu.SMEM),
            pl.BlockSpec(memory_space=pltpu.SMEM),
        ],
        out_shape=[
            jax.ShapeDtypeStruct((rows, K), jnp.int32),
            jax.ShapeDtypeStruct((rows // TB,), jnp.float32),
            jax.ShapeDtypeStruct((rows // TB,), jnp.float32),
        ],
        scratch_shapes=[pltpu.VMEM((TB, P), jnp.float32)],
        compiler_params=pltpu.CompilerParams(
            dimension_semantics=("arbitrary", "arbitrary")),
    )(q, keys, kn)


_SC_NC = 2
_SC_NS = 16
_NW = _SC_NC * _SC_NS
_CB = 8


def _sc_gather_mean(idx_flat, ppg2d, prompt, base_row, rows):
    _PER_W = rows // _NW
    _NCHUNK = _PER_W // _CB
    mesh = plsc.VectorSubcoreMesh(core_axis_name="c", subcore_axis_name="s")

    @functools.partial(
        pl.kernel,
        mesh=mesh,
        out_type=jax.ShapeDtypeStruct((rows, D), jnp.float32),
        scratch_types=[
            pltpu.VMEM((_CB * K,), jnp.int32),
            pltpu.VMEM((_CB * K, D), jnp.float32),
            pltpu.VMEM((_CB, D), jnp.float32),
            pltpu.VMEM((2, _CB, D), jnp.float32),
            pltpu.SemaphoreType.DMA,
            pltpu.SemaphoreType.DMA,
            pltpu.SemaphoreType.DMA,
        ],
    )
    def sc_kernel(idx_hbm, ppg_hbm, prompt_hbm, out_hbm,
                  idx_v, rows_v, ppg_v, out_v, semg, semg2, semo):
        wid = lax.axis_index("s") * _SC_NC + lax.axis_index("c")
        out_handles = {}

        H = _CB // 2

        def chunk(c):
            b = c % 2
            base = wid * _PER_W + c * _CB
            pltpu.sync_copy(idx_hbm.at[pl.ds(base * K, _CB * K)], idx_v)
            g0 = pltpu.async_copy(prompt_hbm.at[idx_v.at[pl.ds(0, H * K)]],
                                  rows_v.at[pl.ds(0, H * K)], semg)
            g1 = pltpu.async_copy(
                prompt_hbm.at[idx_v.at[pl.ds(H * K, H * K)]],
                rows_v.at[pl.ds(H * K, H * K)], semg2)
            pltpu.sync_copy(ppg_hbm.at[pl.ds(base_row + base, _CB)], ppg_v)

            def half_body(r0):
                def dbody(dd, c2):
                    off = dd * 16
                    for r in range(r0, r0 + H):
                        acc = rows_v[r * K + 0, pl.ds(off, 16)]
                        for k in range(1, K):
                            acc = acc + rows_v[r * K + k, pl.ds(off, 16)]
                        out_v[b, r, pl.ds(off, 16)] = (
                            ppg_v[r, pl.ds(off, 16)] + acc * (1.0 / K))
                    return c2

                lax.fori_loop(0, D // 16, dbody, 0)

            g0.wait()
            half_body(0)
            g1.wait()
            half_body(H)
            out_handles[c] = pltpu.async_copy(
                out_v.at[b], out_hbm.at[pl.ds(base, _CB)], semo)

        for c in range(_NCHUNK):
            if c >= 2:
                out_handles[c - 2].wait()
            chunk(c)
        out_handles[_NCHUNK - 2].wait()
        out_handles[_NCHUNK - 1].wait()

    return sc_kernel(idx_flat, ppg2d, prompt)


def kernel(ppg, mode, group_labels, keys, prompt, group_table):
    q = ppg[:, 0, :]
    kn = _key_norms(keys)
    parts, ents, ssums = [], [], []
    for s in range(B // SLAB):
        idx_s, ent_s, ssum_s = _tc_scores_topk(q, keys, kn, s * SLAB, SLAB)
        parts.append(_sc_gather_mean(idx_s.reshape(SLAB * K), q, prompt,
                                     s * SLAB, SLAB))
        ents.append(ent_s)
        ssums.append(ssum_s)
    prompted = jnp.concatenate(parts, axis=0)[:, None, :]
    score_mean = jnp.sum(jnp.stack(ssums)) / (B * K)
    entropy = jnp.sum(jnp.stack(ents)) / B
    return (prompted, score_mean, entropy)
